# baseline (speedup 1.0000x reference)
import functools

import jax
import jax.numpy as jnp
from jax import lax
from jax.experimental import pallas as pl
from jax.experimental.pallas import tpu as pltpu
from jax.experimental.pallas import tpu_sc as plsc

_B = 1024
_D = 16
_NBITS = 32
_STAGE = 128


def _gather_call(token_ids, weight_float):
    info = plsc.get_sparse_core_info()
    nc, ns, _ = info.num_cores, info.num_subcores, info.num_lanes
    nw = nc * ns
    bpw = _B // nw

    mesh = plsc.VectorSubcoreMesh(core_axis_name="c", subcore_axis_name="s")

    @functools.partial(
        pl.kernel,
        mesh=mesh,
        out_type=jax.ShapeDtypeStruct((_B, _STAGE), jnp.float32),
        scratch_types=[
            pltpu.VMEM((bpw,), jnp.int32),
            pltpu.VMEM((bpw, _D), jnp.float32),
            pltpu.SemaphoreType.DMA,
        ],
        compiler_params=pltpu.CompilerParams(
            needs_layout_passes=False, use_tc_tiling_on_sc=False),
    )
    def gather_rows(ids_hbm, table_hbm, out_hbm, idx_v, rows_v, sem):
        wid = lax.axis_index("s") * nc + lax.axis_index("c")
        base = wid * bpw
        pltpu.sync_copy(ids_hbm.at[pl.ds(base, bpw)], idx_v)
        pltpu.async_copy(table_hbm.at[idx_v], rows_v, sem).wait()
        pltpu.sync_copy(rows_v, out_hbm.at[pl.ds(base, bpw), pl.ds(0, _D)])

    return gather_rows(token_ids, weight_float)


def _expand_call(rows):
    def body(rows_ref, out_ref):
        bits = lax.bitcast_convert_type(rows_ref[...], jnp.int32)
        shifts = 31 - lax.broadcasted_iota(jnp.int32, (1, _NBITS), 1)
        for d in range(_D):
            col = bits[:, d:d + 1]
            out_ref[:, d * _NBITS:(d + 1) * _NBITS] = (
                ((col >> shifts) & 1).astype(jnp.float32))

    return pl.pallas_call(
        body,
        out_shape=jax.ShapeDtypeStruct((_B, _D * _NBITS), jnp.float32),
    )(rows)


def kernel(token_ids, weight_float):
    rows = _gather_call(token_ids.astype(jnp.int32),
                        weight_float.astype(jnp.float32))
    out = _expand_call(rows)
    return out.reshape(_B, _D, _NBITS)


# speedup vs baseline: 1.0902x; 1.0902x over previous
import functools

import jax
import jax.numpy as jnp
from jax import lax
from jax.experimental import pallas as pl
from jax.experimental.pallas import tpu as pltpu
from jax.experimental.pallas import tpu_sc as plsc

_B = 1024
_D = 16
_NBITS = 32
_STAGE = 128


def _gather_call(token_ids, weight_float):
    info = plsc.get_sparse_core_info()
    nc, ns, _ = info.num_cores, info.num_subcores, info.num_lanes
    nw = nc * ns
    bpw = _B // nw

    mesh = plsc.VectorSubcoreMesh(core_axis_name="c", subcore_axis_name="s")

    @functools.partial(
        pl.kernel,
        mesh=mesh,
        out_type=jax.ShapeDtypeStruct((_B, _STAGE), jnp.float32),
        scratch_types=[
            pltpu.VMEM((bpw,), jnp.int32),
            pltpu.VMEM((bpw, _D), jnp.float32),
            pltpu.SemaphoreType.DMA,
        ],
        compiler_params=pltpu.CompilerParams(
            needs_layout_passes=False, use_tc_tiling_on_sc=False),
    )
    def gather_rows(ids_hbm, table_hbm, out_hbm, idx_v, rows_v, sem):
        wid = lax.axis_index("s") * nc + lax.axis_index("c")
        base = wid * bpw
        pltpu.sync_copy(ids_hbm.at[pl.ds(base, bpw)], idx_v)
        pltpu.async_copy(table_hbm.at[idx_v], rows_v, sem).wait()
        pltpu.sync_copy(rows_v, out_hbm.at[pl.ds(base, bpw), pl.ds(0, _D)])

    return gather_rows(token_ids, weight_float)


def _expand_call(rows):
    def body(rows_ref, out_ref):
        bits = lax.bitcast_convert_type(rows_ref[:, :_D], jnp.int32)
        j = lax.broadcasted_iota(jnp.int32, (_B, _D * _NBITS), 1)
        spread = jnp.take_along_axis(bits, j >> 5, axis=1)
        out_ref[...] = ((spread >> (31 - (j & 31))) & 1).astype(jnp.float32)

    return pl.pallas_call(
        body,
        out_shape=jax.ShapeDtypeStruct((_B, _D * _NBITS), jnp.float32),
    )(rows)


def kernel(token_ids, weight_float):
    rows = _gather_call(token_ids.astype(jnp.int32),
                        weight_float.astype(jnp.float32))
    out = _expand_call(rows)
    return out.reshape(_B, _D, _NBITS)


# speedup vs baseline: 1.1591x; 1.0632x over previous
import functools

import jax
import jax.numpy as jnp
from jax import lax
from jax.experimental import pallas as pl
from jax.experimental.pallas import tpu as pltpu
from jax.experimental.pallas import tpu_sc as plsc

_B = 1024
_D = 16
_NBITS = 32


def _spike_embed_call(token_ids, weight_float):
    info = plsc.get_sparse_core_info()
    nc, ns, nl = info.num_cores, info.num_subcores, info.num_lanes
    nw = nc * ns
    bpw = _B // nw

    mesh = plsc.VectorSubcoreMesh(core_axis_name="c", subcore_axis_name="s")

    @functools.partial(
        pl.kernel,
        mesh=mesh,
        out_type=jax.ShapeDtypeStruct((_B, _D * _NBITS), jnp.float32),
        scratch_types=[
            pltpu.VMEM((bpw,), jnp.int32),
            pltpu.VMEM((bpw, _D), jnp.float32),
            pltpu.VMEM((bpw, _D * _NBITS), jnp.float32),
            pltpu.SemaphoreType.DMA,
        ],
        compiler_params=pltpu.CompilerParams(
            needs_layout_passes=False, use_tc_tiling_on_sc=False),
    )
    def spike_embed(ids_hbm, table_hbm, out_hbm, idx_v, rows_v, out_v, sem):
        wid = lax.axis_index("s") * nc + lax.axis_index("c")
        base = wid * bpw
        pltpu.sync_copy(ids_hbm.at[pl.ds(base, bpw)], idx_v)
        pltpu.async_copy(table_hbm.at[idx_v], rows_v, sem).wait()

        lane = lax.iota(jnp.int32, nl)
        shifts = [31 - lane, 15 - lane]
        zeros = jnp.zeros((nl,), jnp.int32)

        @plsc.parallel_loop(0, bpw, 1, unroll=2)
        def token_body(t):
            row = plsc.bitcast(rows_v[t], jnp.int32)
            for d in range(_D):
                word = zeros + row[d]
                for h in range(2):
                    bits = ((word >> shifts[h]) & 1).astype(jnp.float32)
                    out_v[t, pl.ds(d * _NBITS + h * nl, nl)] = bits

        pltpu.sync_copy(out_v, out_hbm.at[pl.ds(base, bpw)])

    return spike_embed(token_ids, weight_float)


def kernel(token_ids, weight_float):
    out = _spike_embed_call(token_ids.astype(jnp.int32),
                            weight_float.astype(jnp.float32))
    return out.reshape(_B, _D, _NBITS)
